# Initial kernel scaffold; baseline (speedup 1.0000x reference)
import jax
import jax.numpy as jnp
from jax.experimental import pallas as pl


def kernel(images, outputs, prefix):
    raise NotImplementedError("write your pallas kernel here")


# speedup vs baseline: 3.2848x; 3.2848x over previous
import functools

import jax
import jax.numpy as jnp
from jax import lax
from jax.experimental import pallas as pl
from jax.experimental.pallas import tpu as pltpu
from jax.experimental.pallas import tpu_sc as plsc

S = 7
NCELL = S * S
D = 30
BATCH = 64
ROW_PAD = 1536
DET_PAD = 320
CONF_THRES = 0.5
NMS_THRES = 0.7
GRID = 64.0
WIMG = 448.0
NEG_INF = float("-inf")

_mesh = plsc.VectorSubcoreMesh(core_axis_name="c", subcore_axis_name="s")


@functools.partial(
    pl.kernel,
    out_type=(
        jax.ShapeDtypeStruct((BATCH, DET_PAD), jnp.float32),
        jax.ShapeDtypeStruct((BATCH, 64), jnp.int32),
        jax.ShapeDtypeStruct((BATCH, 64), jnp.int32),
    ),
    mesh=_mesh,
    compiler_params=pltpu.CompilerParams(needs_layout_passes=False),
    scratch_types=[
        pltpu.VMEM((ROW_PAD,), jnp.float32),
        pltpu.VMEM((64,), jnp.float32),
        pltpu.VMEM((64,), jnp.float32),
        pltpu.VMEM((64,), jnp.float32),
        pltpu.VMEM((64,), jnp.float32),
        pltpu.VMEM((64,), jnp.float32),
        pltpu.VMEM((64,), jnp.float32),
        pltpu.VMEM((64,), jnp.int32),
        pltpu.VMEM((64,), jnp.float32),
        pltpu.VMEM((64,), jnp.float32),
        pltpu.VMEM((64,), jnp.float32),
        pltpu.VMEM((64,), jnp.float32),
        pltpu.VMEM((64,), jnp.float32),
        pltpu.VMEM((64,), jnp.int32),
        pltpu.VMEM((DET_PAD,), jnp.float32),
    ],
)
def _yolo_sc(outp_hbm, det_hbm, cls_hbm, keep_hbm,
             buf, x1u, y1u, x2u, y2u, cfa, cpa, cia,
             x1o, y1o, x2o, y2o, ara, kpa, db):
    wid = lax.axis_index("s") * 2 + lax.axis_index("c")
    lane = jnp.arange(16, dtype=jnp.int32)

    for k in range(2):
        img = wid * 2 + k
        pltpu.sync_copy(outp_hbm.at[img], buf)

        sm_chunks = []
        for c in range(4):
            g = lane + c * 16
            gc = jnp.minimum(g, NCELL - 1)
            base = gc * D
            conf0 = plsc.load_gather(buf, [base + 4])
            conf1 = plsc.load_gather(buf, [base + 9])
            use1 = conf1 > conf0
            conf = jnp.maximum(conf0, conf1)
            boff = base + jnp.where(use1, 5, 0)
            bx = plsc.load_gather(buf, [boff])
            by = plsc.load_gather(buf, [boff + 1])
            bw = plsc.load_gather(buf, [boff + 2])
            bh = plsc.load_gather(buf, [boff + 3])
            colf = (gc % S).astype(jnp.float32)
            rowf = (gc // S).astype(jnp.float32)
            cx = (bx + colf) * GRID
            cy = (by + rowf) * GRID
            w = bw * WIMG
            h = bh * WIMG
            x1 = cx - w * 0.5
            y1 = cy - h * 0.5
            x2 = cx + w * 0.5
            y2 = cy + h * 0.5
            best = plsc.load_gather(buf, [base + 10])
            bidx = jnp.zeros((16,), jnp.int32)
            for kk in range(1, 20):
                v = plsc.load_gather(buf, [base + 10 + kk])
                bidx = jnp.where(v > best, kk, bidx)
                best = jnp.maximum(best, v)
            valid = (conf > CONF_THRES) & (g < NCELL)
            offv = bidx.astype(jnp.float32) * (2.0 * WIMG + 1.0)
            xo1 = x1 + offv
            xo2 = x2 + offv
            yo1 = y1 + offv
            yo2 = y2 + offv
            area = jnp.maximum(xo2 - xo1, 0.0) * jnp.maximum(yo2 - yo1, 0.0)
            smv = jnp.where(valid, conf, NEG_INF)
            sl = pl.ds(c * 16, 16)
            x1u[sl] = x1
            y1u[sl] = y1
            x2u[sl] = x2
            y2u[sl] = y2
            cfa[sl] = conf
            cpa[sl] = best
            cia[sl] = bidx
            x1o[sl] = xo1
            y1o[sl] = yo1
            x2o[sl] = xo2
            y2o[sl] = yo2
            ara[sl] = area
            kpa[sl] = valid.astype(jnp.int32)
            sm_chunks.append(smv)

        def nms_body(_, sm):
            s0, s1, s2, s3 = sm
            mx = jnp.max(jnp.maximum(jnp.maximum(s0, s1), jnp.maximum(s2, s3)))
            cands = [
                jnp.where(s_c == mx, lane + c * 16, 999)
                for c, s_c in enumerate((s0, s1, s2, s3))
            ]
            jstar = jnp.min(jnp.minimum(jnp.minimum(cands[0], cands[1]),
                                        jnp.minimum(cands[2], cands[3])))
            jv = jnp.full((16,), jstar, jnp.int32)
            x1c = plsc.load_gather(x1o, [jv])
            y1c = plsc.load_gather(y1o, [jv])
            x2c = plsc.load_gather(x2o, [jv])
            y2c = plsc.load_gather(y2o, [jv])
            arc = plsc.load_gather(ara, [jv])
            kcur = plsc.load_gather(kpa, [jv]) != 0
            new_sm = []
            for c, s_c in enumerate((s0, s1, s2, s3)):
                idxs = lane + c * 16
                unproc = (s_c != NEG_INF) & (idxs != jstar)
                sl = pl.ds(c * 16, 16)
                xx1 = jnp.maximum(x1o[sl], x1c)
                yy1 = jnp.maximum(y1o[sl], y1c)
                xx2 = jnp.minimum(x2o[sl], x2c)
                yy2 = jnp.minimum(y2o[sl], y2c)
                inter = (jnp.maximum(xx2 - xx1, 0.0)
                         * jnp.maximum(yy2 - yy1, 0.0))
                union = ara[sl] + arc - inter
                iou = inter / jnp.maximum(union, 1e-9)
                sup = (iou > NMS_THRES) & unproc & kcur
                kpa[sl] = jnp.where(sup, 0, kpa[sl])
                new_sm.append(jnp.where(idxs == jstar, NEG_INF, s_c))
            return tuple(new_sm)

        lax.fori_loop(0, NCELL, nms_body, tuple(sm_chunks))

        for c in range(4):
            sl = pl.ds(c * 16, 16)
            g = lane + c * 16
            m49 = g < NCELL
            kv = kpa[sl] != 0
            didx = g * 6
            plsc.store_scatter(db, [didx], jnp.where(kv, x1u[sl], 0.0), mask=m49)
            plsc.store_scatter(db, [didx + 1], jnp.where(kv, y1u[sl], 0.0), mask=m49)
            plsc.store_scatter(db, [didx + 2], jnp.where(kv, x2u[sl], 0.0), mask=m49)
            plsc.store_scatter(db, [didx + 3], jnp.where(kv, y2u[sl], 0.0), mask=m49)
            plsc.store_scatter(db, [didx + 4], jnp.where(kv, cfa[sl], 0.0), mask=m49)
            plsc.store_scatter(db, [didx + 5], jnp.where(kv, cpa[sl], 0.0), mask=m49)
        pltpu.sync_copy(db, det_hbm.at[img])
        pltpu.sync_copy(cia, cls_hbm.at[img])
        pltpu.sync_copy(kpa, keep_hbm.at[img])


def kernel(images, outputs, prefix=0):
    del images, prefix
    outp = jnp.pad(outputs, ((0, 0), (0, ROW_PAD - S * S * D)))
    det_p, cls_p, keep_p = _yolo_sc(outp)
    det = det_p[:, : NCELL * 6].reshape(BATCH, NCELL, 6)
    return det, cls_p[:, :NCELL], keep_p[:, :NCELL] != 0
